# baseline (speedup 1.0000x reference)
import functools

import jax
import jax.numpy as jnp
from jax import lax
from jax.experimental import pallas as pl
from jax.experimental.pallas import tpu as pltpu
from jax.experimental.pallas import tpu_sc as plsc

_NE = 64
_NC = 2
_NS = 16
_NW = _NC * _NS
_BT = 512
_CH = 128


def _prep_body(C, x_ref, Win_ref, bin_ref, Wg_ref,
               h_ref, sidx_ref, gidx_ref, gk_ref, cnt_ref):
    i = pl.program_id(0)

    @pl.when(i == 0)
    def _():
        cnt_ref[...] = jnp.zeros_like(cnt_ref)

    x = x_ref[...]
    h = jnp.dot(x, Win_ref[...], preferred_element_type=jnp.float32) + bin_ref[...]

    logits = jnp.dot(h, Wg_ref[...], preferred_element_type=jnp.float32)
    m = jnp.max(logits, axis=-1, keepdims=True)
    s = jnp.sum(jnp.exp(logits - m), axis=-1, keepdims=True)
    gate = 1.0 / s
    lane = lax.broadcasted_iota(jnp.int32, logits.shape, 1)
    idx = jnp.min(jnp.where(logits == m, lane, _NE), axis=-1, keepdims=True)
    onehot = (lane == idx).astype(jnp.float32)

    bt = logits.shape[0]
    tril = (lax.broadcasted_iota(jnp.int32, (bt, bt), 0) >=
            lax.broadcasted_iota(jnp.int32, (bt, bt), 1)).astype(jnp.float32)
    cs = jnp.dot(tril, onehot, preferred_element_type=jnp.float32)
    prev = cnt_ref[...]
    pos = jnp.sum(onehot * (cs - 1.0 + prev), axis=-1, keepdims=True)
    cnt_ref[...] = prev + cs[bt - 1:bt, :]
    ipos = pos.astype(jnp.int32)
    keep = ipos < C

    h_ref[...] = h.astype(jnp.bfloat16)
    sidx_ref[...] = jnp.where(keep, idx * C + ipos, _NE * C)
    gidx_ref[...] = idx * C + jnp.minimum(ipos, C - 1)
    gk_ref[...] = gate * keep.astype(jnp.float32)


def _res_body(h_ref, gk_ref, Wr1_ref, br1_ref, Wr2_ref, br2_ref,
              Wc_ref, bc_ref, resw_ref, scale_ref):
    h = h_ref[...].astype(jnp.float32)
    t1 = jnp.maximum(
        jnp.dot(h, Wr1_ref[...], preferred_element_type=jnp.float32)
        + br1_ref[...], 0.0)
    res = (jnp.dot(t1, Wr2_ref[...], preferred_element_type=jnp.float32)
           + br2_ref[...])
    cl = jnp.dot(h, Wc_ref[...], preferred_element_type=jnp.float32) + bc_ref[...]
    ce = jnp.exp(cl - jnp.max(cl, axis=-1, keepdims=True))
    coef = ce / jnp.sum(ce, axis=-1, keepdims=True)
    resw_ref[...] = res * coef[:, 1:2]
    scale_ref[...] = gk_ref[...] * coef[:, 0:1]


def _ffn_body(ein_ref, W1_ref, b1_ref, W2_ref, b2_ref, eout_ref):
    a = jnp.maximum(
        jnp.dot(ein_ref[...].astype(jnp.float32), W1_ref[0],
                preferred_element_type=jnp.float32) + b1_ref[0], 0.0)
    eout_ref[...] = (
        jnp.dot(a, W2_ref[0], preferred_element_type=jnp.float32)
        + b2_ref[0]).astype(jnp.bfloat16)


def _combine_body(moe_ref, scale_ref, resw_ref, Wout_ref, bout_ref, o_ref):
    mix = moe_ref[...].astype(jnp.float32) * scale_ref[...] + resw_ref[...]
    o_ref[...] = (
        jnp.dot(mix, Wout_ref[...], preferred_element_type=jnp.float32)
        + bout_ref[...])


def _sc_scatter(T, d, rows_out):
    per_w = T // _NW
    n_ch = per_w // _CH
    mesh = plsc.VectorSubcoreMesh(core_axis_name="c", subcore_axis_name="s")

    @functools.partial(
        pl.kernel,
        out_type=jax.ShapeDtypeStruct((rows_out, d), jnp.int32),
        mesh=mesh,
        scratch_types=[
            pltpu.VMEM((n_ch, _CH), jnp.int32),
            pltpu.VMEM((_CH, d), jnp.int32),
            pltpu.SemaphoreType.DMA,
        ],
    )
    def scat(h_hbm, sidx_hbm, out_hbm, idx_v, rows_v, sem):
        wid = lax.axis_index("s") * _NC + lax.axis_index("c")
        base = wid * per_w
        for j in range(n_ch):
            pltpu.sync_copy(sidx_hbm.at[pl.ds(base + j * _CH, _CH)], idx_v.at[j])
        for j in range(n_ch):
            pltpu.async_copy(h_hbm.at[pl.ds(base + j * _CH, _CH)], rows_v, sem).wait()
            pltpu.sync_copy(rows_v, out_hbm.at[idx_v.at[j]])

    return scat


def _sc_gather(T, d, rows_in):
    per_w = T // _NW
    n_ch = per_w // _CH
    mesh = plsc.VectorSubcoreMesh(core_axis_name="c", subcore_axis_name="s")

    @functools.partial(
        pl.kernel,
        out_type=jax.ShapeDtypeStruct((T, d), jnp.int32),
        mesh=mesh,
        scratch_types=[
            pltpu.VMEM((n_ch, _CH), jnp.int32),
            pltpu.VMEM((_CH, d), jnp.int32),
            pltpu.SemaphoreType.DMA,
        ],
    )
    def gath(eout_hbm, gidx_hbm, moe_hbm, idx_v, rows_v, sem):
        wid = lax.axis_index("s") * _NC + lax.axis_index("c")
        base = wid * per_w
        for j in range(n_ch):
            pltpu.sync_copy(gidx_hbm.at[pl.ds(base + j * _CH, _CH)], idx_v.at[j])
        for j in range(n_ch):
            pltpu.async_copy(eout_hbm.at[idx_v.at[j]], rows_v, sem).wait()
            pltpu.sync_copy(rows_v, moe_hbm.at[pl.ds(base + j * _CH, _CH)])

    return gath


def kernel(x, W_in, b_in, Wg, W1, b1, W2, b2, Wr1, br1, Wr2, br2,
           Wc, bc, W_out, b_out):
    B, S, d = x.shape
    T = B * S
    C = -(-T // _NE)
    nblk = T // _BT
    x2 = x.reshape(T, d)

    prep = pl.pallas_call(
        functools.partial(_prep_body, C),
        grid=(nblk,),
        in_specs=[
            pl.BlockSpec((_BT, d), lambda i: (i, 0)),
            pl.BlockSpec((d, d), lambda i: (0, 0)),
            pl.BlockSpec((1, d), lambda i: (0, 0)),
            pl.BlockSpec((d, _NE), lambda i: (0, 0)),
        ],
        out_specs=[
            pl.BlockSpec((_BT, d), lambda i: (i, 0)),
            pl.BlockSpec((_BT, 1), lambda i: (i, 0)),
            pl.BlockSpec((_BT, 1), lambda i: (i, 0)),
            pl.BlockSpec((_BT, 1), lambda i: (i, 0)),
        ],
        out_shape=[
            jax.ShapeDtypeStruct((T, d), jnp.bfloat16),
            jax.ShapeDtypeStruct((T, 1), jnp.int32),
            jax.ShapeDtypeStruct((T, 1), jnp.int32),
            jax.ShapeDtypeStruct((T, 1), jnp.float32),
        ],
        scratch_shapes=[pltpu.VMEM((1, _NE), jnp.float32)],
    )
    h, sidx, gidx, gk = prep(x2, W_in, b_in.reshape(1, d), Wg)

    def _to_words(a):
        n = a.shape[0]
        return lax.bitcast_convert_type(
            a.reshape(n, d // 2, 2), jnp.int32)

    def _to_bf16(a):
        n = a.shape[0]
        return lax.bitcast_convert_type(a, jnp.bfloat16).reshape(n, d)

    rows = (_NE + 1) * C
    ein = _to_bf16(_sc_scatter(T, d // 2, rows)(_to_words(h), sidx.reshape(T)))

    resffn = pl.pallas_call(
        _res_body,
        grid=(nblk,),
        in_specs=[
            pl.BlockSpec((_BT, d), lambda i: (i, 0)),
            pl.BlockSpec((_BT, 1), lambda i: (i, 0)),
            pl.BlockSpec((d, d), lambda i: (0, 0)),
            pl.BlockSpec((1, d), lambda i: (0, 0)),
            pl.BlockSpec((d, d), lambda i: (0, 0)),
            pl.BlockSpec((1, d), lambda i: (0, 0)),
            pl.BlockSpec((d, 2), lambda i: (0, 0)),
            pl.BlockSpec((1, 2), lambda i: (0, 0)),
        ],
        out_specs=[
            pl.BlockSpec((_BT, d), lambda i: (i, 0)),
            pl.BlockSpec((_BT, 1), lambda i: (i, 0)),
        ],
        out_shape=[
            jax.ShapeDtypeStruct((T, d), jnp.float32),
            jax.ShapeDtypeStruct((T, 1), jnp.float32),
        ],
    )
    resw, scale = resffn(h, gk, Wr1, br1.reshape(1, d), Wr2,
                         br2.reshape(1, d), Wc, bc.reshape(1, 2))

    ffn = pl.pallas_call(
        _ffn_body,
        grid=(_NE,),
        in_specs=[
            pl.BlockSpec((C, d), lambda e: (e, 0)),
            pl.BlockSpec((1, d, d), lambda e: (e, 0, 0)),
            pl.BlockSpec((1, 1, d), lambda e: (e, 0, 0)),
            pl.BlockSpec((1, d, d), lambda e: (e, 0, 0)),
            pl.BlockSpec((1, 1, d), lambda e: (e, 0, 0)),
        ],
        out_specs=pl.BlockSpec((C, d), lambda e: (e, 0)),
        out_shape=jax.ShapeDtypeStruct((_NE * C, d), jnp.bfloat16),
    )
    eout = ffn(ein, W1, b1.reshape(_NE, 1, d), W2, b2.reshape(_NE, 1, d))

    moe = _to_bf16(_sc_gather(T, d // 2, _NE * C)(_to_words(eout),
                                                  gidx.reshape(T)))

    combine = pl.pallas_call(
        _combine_body,
        grid=(nblk,),
        in_specs=[
            pl.BlockSpec((_BT, d), lambda i: (i, 0)),
            pl.BlockSpec((_BT, 1), lambda i: (i, 0)),
            pl.BlockSpec((_BT, d), lambda i: (i, 0)),
            pl.BlockSpec((d, d), lambda i: (0, 0)),
            pl.BlockSpec((1, d), lambda i: (0, 0)),
        ],
        out_specs=pl.BlockSpec((_BT, d), lambda i: (i, 0)),
        out_shape=jax.ShapeDtypeStruct((T, d), jnp.float32),
    )
    out = combine(moe, scale, resw, W_out, b_out.reshape(1, d))
    return out.reshape(B, S, d)


# speedup vs baseline: 2.8383x; 2.8383x over previous
import functools

import jax
import jax.numpy as jnp
from jax import lax
from jax.experimental import pallas as pl
from jax.experimental.pallas import tpu as pltpu
from jax.experimental.pallas import tpu_sc as plsc

_NE = 64
_NC = 2
_NS = 16
_NW = _NC * _NS
_BT = 512
_CH = 128


def _prep_body(C, x_ref, Win_ref, bin_ref, Wg_ref,
               h_ref, sidx_ref, gidx_ref, gk_ref, cnt_ref):
    i = pl.program_id(0)

    @pl.when(i == 0)
    def _():
        cnt_ref[...] = jnp.zeros_like(cnt_ref)

    x = x_ref[...]
    h = jnp.dot(x, Win_ref[...], preferred_element_type=jnp.float32) + bin_ref[...]

    logits = jnp.dot(h, Wg_ref[...], preferred_element_type=jnp.float32)
    m = jnp.max(logits, axis=-1, keepdims=True)
    s = jnp.sum(jnp.exp(logits - m), axis=-1, keepdims=True)
    gate = 1.0 / s
    lane = lax.broadcasted_iota(jnp.int32, logits.shape, 1)
    idx = jnp.min(jnp.where(logits == m, lane, _NE), axis=-1, keepdims=True)
    onehot = (lane == idx).astype(jnp.float32)

    bt = logits.shape[0]
    tril = (lax.broadcasted_iota(jnp.int32, (bt, bt), 0) >=
            lax.broadcasted_iota(jnp.int32, (bt, bt), 1)).astype(jnp.float32)
    cs = jnp.dot(tril, onehot, preferred_element_type=jnp.float32)
    prev = cnt_ref[...]
    pos = jnp.sum(onehot * (cs - 1.0 + prev), axis=-1, keepdims=True)
    cnt_ref[...] = prev + cs[bt - 1:bt, :]
    ipos = pos.astype(jnp.int32)
    keep = ipos < C

    h_ref[...] = h
    sidx_ref[...] = jnp.where(keep, idx * C + ipos, _NE * C)
    gidx_ref[...] = idx * C + jnp.minimum(ipos, C - 1)
    gk_ref[...] = gate * keep.astype(jnp.float32)


def _res_body(h_ref, gk_ref, Wr1_ref, br1_ref, Wr2_ref, br2_ref,
              Wc_ref, bc_ref, resw_ref, scale_ref):
    h = h_ref[...]
    t1 = jnp.maximum(
        jnp.dot(h, Wr1_ref[...], preferred_element_type=jnp.float32)
        + br1_ref[...], 0.0)
    res = (jnp.dot(t1, Wr2_ref[...], preferred_element_type=jnp.float32)
           + br2_ref[...])
    cl = jnp.dot(h, Wc_ref[...], preferred_element_type=jnp.float32) + bc_ref[...]
    ce = jnp.exp(cl - jnp.max(cl, axis=-1, keepdims=True))
    coef = ce / jnp.sum(ce, axis=-1, keepdims=True)
    resw_ref[...] = res * coef[:, 1:2]
    scale_ref[...] = gk_ref[...] * coef[:, 0:1]


def _ffn_body(ein_ref, W1_ref, b1_ref, W2_ref, b2_ref, eout_ref):
    a = jnp.maximum(
        jnp.dot(ein_ref[...], W1_ref[0], preferred_element_type=jnp.float32)
        + b1_ref[0], 0.0)
    eout_ref[...] = (
        jnp.dot(a, W2_ref[0], preferred_element_type=jnp.float32) + b2_ref[0])


def _combine_body(moe_ref, scale_ref, resw_ref, Wout_ref, bout_ref, o_ref):
    mix = moe_ref[...] * scale_ref[...] + resw_ref[...]
    o_ref[...] = (
        jnp.dot(mix, Wout_ref[...], preferred_element_type=jnp.float32)
        + bout_ref[...])


def _sc_scatter(T, d, rows_out):
    per_w = T // _NW
    n_ch = per_w // _CH
    mesh = plsc.VectorSubcoreMesh(core_axis_name="c", subcore_axis_name="s")

    @functools.partial(
        pl.kernel,
        out_type=jax.ShapeDtypeStruct((rows_out, d), jnp.float32),
        mesh=mesh,
        scratch_types=[
            pltpu.VMEM((n_ch, _CH), jnp.int32),
            pltpu.VMEM((_CH, d), jnp.float32),
            pltpu.SemaphoreType.DMA,
        ],
    )
    def scat(h_hbm, sidx_hbm, out_hbm, idx_v, rows_v, sem):
        wid = lax.axis_index("s") * _NC + lax.axis_index("c")
        base = wid * per_w
        for j in range(n_ch):
            pltpu.sync_copy(sidx_hbm.at[pl.ds(base + j * _CH, _CH)], idx_v.at[j])
        for j in range(n_ch):
            pltpu.async_copy(h_hbm.at[pl.ds(base + j * _CH, _CH)], rows_v, sem).wait()
            pltpu.sync_copy(rows_v, out_hbm.at[idx_v.at[j]])

    return scat


def _sc_gather(T, d, rows_in):
    per_w = T // _NW
    n_ch = per_w // _CH
    mesh = plsc.VectorSubcoreMesh(core_axis_name="c", subcore_axis_name="s")

    @functools.partial(
        pl.kernel,
        out_type=jax.ShapeDtypeStruct((T, d), jnp.float32),
        mesh=mesh,
        scratch_types=[
            pltpu.VMEM((n_ch, _CH), jnp.int32),
            pltpu.VMEM((_CH, d), jnp.float32),
            pltpu.SemaphoreType.DMA,
        ],
    )
    def gath(eout_hbm, gidx_hbm, moe_hbm, idx_v, rows_v, sem):
        wid = lax.axis_index("s") * _NC + lax.axis_index("c")
        base = wid * per_w
        for j in range(n_ch):
            pltpu.sync_copy(gidx_hbm.at[pl.ds(base + j * _CH, _CH)], idx_v.at[j])
        for j in range(n_ch):
            pltpu.async_copy(eout_hbm.at[idx_v.at[j]], rows_v, sem).wait()
            pltpu.sync_copy(rows_v, moe_hbm.at[pl.ds(base + j * _CH, _CH)])

    return gath


def kernel(x, W_in, b_in, Wg, W1, b1, W2, b2, Wr1, br1, Wr2, br2,
           Wc, bc, W_out, b_out):
    B, S, d = x.shape
    T = B * S
    C = -(-T // _NE)
    nblk = T // _BT
    x2 = x.reshape(T, d)

    prep = pl.pallas_call(
        functools.partial(_prep_body, C),
        grid=(nblk,),
        in_specs=[
            pl.BlockSpec((_BT, d), lambda i: (i, 0)),
            pl.BlockSpec((d, d), lambda i: (0, 0)),
            pl.BlockSpec((1, d), lambda i: (0, 0)),
            pl.BlockSpec((d, _NE), lambda i: (0, 0)),
        ],
        out_specs=[
            pl.BlockSpec((_BT, d), lambda i: (i, 0)),
            pl.BlockSpec((_BT, 1), lambda i: (i, 0)),
            pl.BlockSpec((_BT, 1), lambda i: (i, 0)),
            pl.BlockSpec((_BT, 1), lambda i: (i, 0)),
        ],
        out_shape=[
            jax.ShapeDtypeStruct((T, d), jnp.float32),
            jax.ShapeDtypeStruct((T, 1), jnp.int32),
            jax.ShapeDtypeStruct((T, 1), jnp.int32),
            jax.ShapeDtypeStruct((T, 1), jnp.float32),
        ],
        scratch_shapes=[pltpu.VMEM((1, _NE), jnp.float32)],
    )
    h, sidx, gidx, gk = prep(x2, W_in, b_in.reshape(1, d), Wg)

    rows = (_NE + 1) * C
    ein = _sc_scatter(T, d, rows)(h, sidx.reshape(T))

    resffn = pl.pallas_call(
        _res_body,
        grid=(nblk,),
        in_specs=[
            pl.BlockSpec((_BT, d), lambda i: (i, 0)),
            pl.BlockSpec((_BT, 1), lambda i: (i, 0)),
            pl.BlockSpec((d, d), lambda i: (0, 0)),
            pl.BlockSpec((1, d), lambda i: (0, 0)),
            pl.BlockSpec((d, d), lambda i: (0, 0)),
            pl.BlockSpec((1, d), lambda i: (0, 0)),
            pl.BlockSpec((d, 2), lambda i: (0, 0)),
            pl.BlockSpec((1, 2), lambda i: (0, 0)),
        ],
        out_specs=[
            pl.BlockSpec((_BT, d), lambda i: (i, 0)),
            pl.BlockSpec((_BT, 1), lambda i: (i, 0)),
        ],
        out_shape=[
            jax.ShapeDtypeStruct((T, d), jnp.float32),
            jax.ShapeDtypeStruct((T, 1), jnp.float32),
        ],
    )
    resw, scale = resffn(h, gk, Wr1, br1.reshape(1, d), Wr2,
                         br2.reshape(1, d), Wc, bc.reshape(1, 2))

    ffn = pl.pallas_call(
        _ffn_body,
        grid=(_NE,),
        in_specs=[
            pl.BlockSpec((C, d), lambda e: (e, 0)),
            pl.BlockSpec((1, d, d), lambda e: (e, 0, 0)),
            pl.BlockSpec((1, 1, d), lambda e: (e, 0, 0)),
            pl.BlockSpec((1, d, d), lambda e: (e, 0, 0)),
            pl.BlockSpec((1, 1, d), lambda e: (e, 0, 0)),
        ],
        out_specs=pl.BlockSpec((C, d), lambda e: (e, 0)),
        out_shape=jax.ShapeDtypeStruct((_NE * C, d), jnp.float32),
    )
    eout = ffn(ein, W1, b1.reshape(_NE, 1, d), W2, b2.reshape(_NE, 1, d))

    moe = _sc_gather(T, d, _NE * C)(eout, gidx.reshape(T))

    combine = pl.pallas_call(
        _combine_body,
        grid=(nblk,),
        in_specs=[
            pl.BlockSpec((_BT, d), lambda i: (i, 0)),
            pl.BlockSpec((_BT, 1), lambda i: (i, 0)),
            pl.BlockSpec((_BT, d), lambda i: (i, 0)),
            pl.BlockSpec((d, d), lambda i: (0, 0)),
            pl.BlockSpec((1, d), lambda i: (0, 0)),
        ],
        out_specs=pl.BlockSpec((_BT, d), lambda i: (i, 0)),
        out_shape=jax.ShapeDtypeStruct((T, d), jnp.float32),
    )
    out = combine(moe, scale, resw, W_out, b_out.reshape(1, d))
    return out.reshape(B, S, d)


# speedup vs baseline: 2.9244x; 1.0303x over previous
import functools

import jax
import jax.numpy as jnp
from jax import lax
from jax.experimental import pallas as pl
from jax.experimental.pallas import tpu as pltpu
from jax.experimental.pallas import tpu_sc as plsc

_NE = 64
_NC = 2
_NS = 16
_NW = _NC * _NS
_BT = 1024
_CH = 128


def _prep_body(C, x_ref, Win_ref, bin_ref, Wg_ref,
               h_ref, sidx_ref, gidx_ref, gk_ref, cnt_ref):
    i = pl.program_id(0)

    @pl.when(i == 0)
    def _():
        cnt_ref[...] = jnp.zeros_like(cnt_ref)

    x = x_ref[...]
    h = jnp.dot(x, Win_ref[...], preferred_element_type=jnp.float32) + bin_ref[...]

    logits = jnp.dot(h, Wg_ref[...], preferred_element_type=jnp.float32)
    m = jnp.max(logits, axis=-1, keepdims=True)
    s = jnp.sum(jnp.exp(logits - m), axis=-1, keepdims=True)
    gate = 1.0 / s
    lane = lax.broadcasted_iota(jnp.int32, logits.shape, 1)
    idx = jnp.min(jnp.where(logits == m, lane, _NE), axis=-1, keepdims=True)
    onehot = (lane == idx).astype(jnp.float32)

    bt = logits.shape[0]
    tril = (lax.broadcasted_iota(jnp.int32, (bt, bt), 0) >=
            lax.broadcasted_iota(jnp.int32, (bt, bt), 1)).astype(jnp.float32)
    cs = jnp.dot(tril, onehot, preferred_element_type=jnp.float32)
    prev = cnt_ref[...]
    pos = jnp.sum(onehot * (cs - 1.0 + prev), axis=-1, keepdims=True)
    cnt_ref[...] = prev + cs[bt - 1:bt, :]
    ipos = pos.astype(jnp.int32)
    keep = ipos < C

    h_ref[...] = h
    sidx_ref[...] = jnp.where(keep, idx * C + ipos, _NE * C)
    gidx_ref[...] = idx * C + jnp.minimum(ipos, C - 1)
    gk_ref[...] = gate * keep.astype(jnp.float32)


def _res_body(h_ref, gk_ref, Wr1_ref, br1_ref, Wr2_ref, br2_ref,
              Wc_ref, bc_ref, resw_ref, scale_ref):
    h = h_ref[...]
    t1 = jnp.maximum(
        jnp.dot(h, Wr1_ref[...], preferred_element_type=jnp.float32)
        + br1_ref[...], 0.0)
    res = (jnp.dot(t1, Wr2_ref[...], preferred_element_type=jnp.float32)
           + br2_ref[...])
    cl = jnp.dot(h, Wc_ref[...], preferred_element_type=jnp.float32) + bc_ref[...]
    ce = jnp.exp(cl - jnp.max(cl, axis=-1, keepdims=True))
    coef = ce / jnp.sum(ce, axis=-1, keepdims=True)
    resw_ref[...] = res * coef[:, 1:2]
    scale_ref[...] = gk_ref[...] * coef[:, 0:1]


def _ffn_body(ein_ref, W1_ref, b1_ref, W2_ref, b2_ref, eout_ref):
    a = jnp.maximum(
        jnp.dot(ein_ref[...], W1_ref[0], preferred_element_type=jnp.float32)
        + b1_ref[0], 0.0)
    eout_ref[...] = (
        jnp.dot(a, W2_ref[0], preferred_element_type=jnp.float32) + b2_ref[0])


def _combine_body(moe_ref, scale_ref, resw_ref, Wout_ref, bout_ref, o_ref):
    mix = moe_ref[...] * scale_ref[...] + resw_ref[...]
    o_ref[...] = (
        jnp.dot(mix, Wout_ref[...], preferred_element_type=jnp.float32)
        + bout_ref[...])


def _sc_scatter(T, d, rows_out):
    per_w = T // _NW
    n_ch = per_w // _CH
    mesh = plsc.VectorSubcoreMesh(core_axis_name="c", subcore_axis_name="s")

    @functools.partial(
        pl.kernel,
        out_type=jax.ShapeDtypeStruct((rows_out, d), jnp.float32),
        mesh=mesh,
        scratch_types=[
            pltpu.VMEM((n_ch, _CH), jnp.int32),
            pltpu.VMEM((_CH, d), jnp.float32),
            pltpu.SemaphoreType.DMA,
        ],
    )
    def scat(h_hbm, sidx_hbm, out_hbm, idx_v, rows_v, sem):
        wid = lax.axis_index("s") * _NC + lax.axis_index("c")
        base = wid * per_w
        for j in range(n_ch):
            pltpu.sync_copy(sidx_hbm.at[pl.ds(base + j * _CH, _CH)], idx_v.at[j])
        for j in range(n_ch):
            pltpu.async_copy(h_hbm.at[pl.ds(base + j * _CH, _CH)], rows_v, sem).wait()
            pltpu.sync_copy(rows_v, out_hbm.at[idx_v.at[j]])

    return scat


def _sc_gather(T, d, rows_in):
    per_w = T // _NW
    n_ch = per_w // _CH
    mesh = plsc.VectorSubcoreMesh(core_axis_name="c", subcore_axis_name="s")

    @functools.partial(
        pl.kernel,
        out_type=jax.ShapeDtypeStruct((T, d), jnp.float32),
        mesh=mesh,
        scratch_types=[
            pltpu.VMEM((n_ch, _CH), jnp.int32),
            pltpu.VMEM((_CH, d), jnp.float32),
            pltpu.SemaphoreType.DMA,
        ],
    )
    def gath(eout_hbm, gidx_hbm, moe_hbm, idx_v, rows_v, sem):
        wid = lax.axis_index("s") * _NC + lax.axis_index("c")
        base = wid * per_w
        for j in range(n_ch):
            pltpu.sync_copy(gidx_hbm.at[pl.ds(base + j * _CH, _CH)], idx_v.at[j])
        for j in range(n_ch):
            pltpu.async_copy(eout_hbm.at[idx_v.at[j]], rows_v, sem).wait()
            pltpu.sync_copy(rows_v, moe_hbm.at[pl.ds(base + j * _CH, _CH)])

    return gath


def kernel(x, W_in, b_in, Wg, W1, b1, W2, b2, Wr1, br1, Wr2, br2,
           Wc, bc, W_out, b_out):
    B, S, d = x.shape
    T = B * S
    C = -(-T // _NE)
    nblk = T // _BT
    x2 = x.reshape(T, d)

    prep = pl.pallas_call(
        functools.partial(_prep_body, C),
        grid=(nblk,),
        in_specs=[
            pl.BlockSpec((_BT, d), lambda i: (i, 0)),
            pl.BlockSpec((d, d), lambda i: (0, 0)),
            pl.BlockSpec((1, d), lambda i: (0, 0)),
            pl.BlockSpec((d, _NE), lambda i: (0, 0)),
        ],
        out_specs=[
            pl.BlockSpec((_BT, d), lambda i: (i, 0)),
            pl.BlockSpec((_BT, 1), lambda i: (i, 0)),
            pl.BlockSpec((_BT, 1), lambda i: (i, 0)),
            pl.BlockSpec((_BT, 1), lambda i: (i, 0)),
        ],
        out_shape=[
            jax.ShapeDtypeStruct((T, d), jnp.float32),
            jax.ShapeDtypeStruct((T, 1), jnp.int32),
            jax.ShapeDtypeStruct((T, 1), jnp.int32),
            jax.ShapeDtypeStruct((T, 1), jnp.float32),
        ],
        scratch_shapes=[pltpu.VMEM((1, _NE), jnp.float32)],
    )
    h, sidx, gidx, gk = prep(x2, W_in, b_in.reshape(1, d), Wg)

    rows = (_NE + 1) * C
    ein = _sc_scatter(T, d, rows)(h, sidx.reshape(T))

    resffn = pl.pallas_call(
        _res_body,
        grid=(nblk,),
        in_specs=[
            pl.BlockSpec((_BT, d), lambda i: (i, 0)),
            pl.BlockSpec((_BT, 1), lambda i: (i, 0)),
            pl.BlockSpec((d, d), lambda i: (0, 0)),
            pl.BlockSpec((1, d), lambda i: (0, 0)),
            pl.BlockSpec((d, d), lambda i: (0, 0)),
            pl.BlockSpec((1, d), lambda i: (0, 0)),
            pl.BlockSpec((d, 2), lambda i: (0, 0)),
            pl.BlockSpec((1, 2), lambda i: (0, 0)),
        ],
        out_specs=[
            pl.BlockSpec((_BT, d), lambda i: (i, 0)),
            pl.BlockSpec((_BT, 1), lambda i: (i, 0)),
        ],
        out_shape=[
            jax.ShapeDtypeStruct((T, d), jnp.float32),
            jax.ShapeDtypeStruct((T, 1), jnp.float32),
        ],
    )
    resw, scale = resffn(h, gk, Wr1, br1.reshape(1, d), Wr2,
                         br2.reshape(1, d), Wc, bc.reshape(1, 2))

    ffn = pl.pallas_call(
        _ffn_body,
        grid=(_NE,),
        in_specs=[
            pl.BlockSpec((C, d), lambda e: (e, 0)),
            pl.BlockSpec((1, d, d), lambda e: (e, 0, 0)),
            pl.BlockSpec((1, 1, d), lambda e: (e, 0, 0)),
            pl.BlockSpec((1, d, d), lambda e: (e, 0, 0)),
            pl.BlockSpec((1, 1, d), lambda e: (e, 0, 0)),
        ],
        out_specs=pl.BlockSpec((C, d), lambda e: (e, 0)),
        out_shape=jax.ShapeDtypeStruct((_NE * C, d), jnp.float32),
    )
    eout = ffn(ein, W1, b1.reshape(_NE, 1, d), W2, b2.reshape(_NE, 1, d))

    moe = _sc_gather(T, d, _NE * C)(eout, gidx.reshape(T))

    combine = pl.pallas_call(
        _combine_body,
        grid=(nblk,),
        in_specs=[
            pl.BlockSpec((_BT, d), lambda i: (i, 0)),
            pl.BlockSpec((_BT, 1), lambda i: (i, 0)),
            pl.BlockSpec((_BT, d), lambda i: (i, 0)),
            pl.BlockSpec((d, d), lambda i: (0, 0)),
            pl.BlockSpec((1, d), lambda i: (0, 0)),
        ],
        out_specs=pl.BlockSpec((_BT, d), lambda i: (i, 0)),
        out_shape=jax.ShapeDtypeStruct((T, d), jnp.float32),
    )
    out = combine(moe, scale, resw, W_out, b_out.reshape(1, d))
    return out.reshape(B, S, d)
